# baseline (device time: 164961 ns/iter reference)
import jax
import jax.numpy as jnp
from jax import lax
from jax.experimental import pallas as pl
from jax.experimental.pallas import tpu as pltpu

Z = 4


def kernel(Q, K, V):
    b, sq, nh, d = Q.shape
    scale = d ** -0.5

    def body(q_ref, k_ref, v_ref, out_ref, kfull, vfull,
             ksend, krecv, vsend, vrecv):
        my_x = lax.axis_index("x")
        my_y = lax.axis_index("y")
        my_z = lax.axis_index("z")
        right = lax.rem(my_z + 1, Z)
        left = lax.rem(my_z + Z - 1, Z)

        barrier_sem = pltpu.get_barrier_semaphore()
        for nbr in (left, right):
            pl.semaphore_signal(
                barrier_sem, inc=1,
                device_id=(my_x, my_y, nbr),
                device_id_type=pl.DeviceIdType.MESH,
            )
        pl.semaphore_wait(barrier_sem, 2)

        kfull[my_z] = k_ref[...]
        vfull[my_z] = v_ref[...]

        for hop in range(Z - 1):
            slot = lax.rem(my_z - hop + Z, Z)
            k_rdma = pltpu.make_async_remote_copy(
                src_ref=kfull.at[slot],
                dst_ref=kfull.at[slot],
                send_sem=ksend.at[hop],
                recv_sem=krecv.at[hop],
                device_id=(my_x, my_y, right),
                device_id_type=pl.DeviceIdType.MESH,
            )
            v_rdma = pltpu.make_async_remote_copy(
                src_ref=vfull.at[slot],
                dst_ref=vfull.at[slot],
                send_sem=vsend.at[hop],
                recv_sem=vrecv.at[hop],
                device_id=(my_x, my_y, right),
                device_id_type=pl.DeviceIdType.MESH,
            )
            k_rdma.start()
            v_rdma.start()
            k_rdma.wait()
            v_rdma.wait()

        for bi in range(b):
            for hi in range(nh):
                q = q_ref[bi, :, hi, :] * scale
                s = jnp.concatenate(
                    [
                        lax.dot_general(
                            q, kfull[z, bi, :, hi, :],
                            (((1,), (1,)), ((), ())),
                            preferred_element_type=jnp.float32,
                        )
                        for z in range(Z)
                    ],
                    axis=1,
                )
                m = jnp.max(s, axis=1, keepdims=True)
                p = jnp.exp(s - m)
                p = p / jnp.sum(p, axis=1, keepdims=True)
                acc = jnp.zeros((sq, d), dtype=jnp.float32)
                for z in range(Z):
                    acc = acc + lax.dot_general(
                        p[:, z * sq:(z + 1) * sq], vfull[z, bi, :, hi, :],
                        (((1,), (0,)), ((), ())),
                        preferred_element_type=jnp.float32,
                    )
                out_ref[bi, :, hi, :] = acc

    return pl.pallas_call(
        body,
        out_shape=jax.ShapeDtypeStruct((b, sq, nh, d), jnp.float32),
        in_specs=[pl.BlockSpec(memory_space=pltpu.VMEM)] * 3,
        out_specs=pl.BlockSpec(memory_space=pltpu.VMEM),
        scratch_shapes=[
            pltpu.VMEM((Z, b, sq, nh, d), jnp.float32),
            pltpu.VMEM((Z, b, sq, nh, d), jnp.float32),
            pltpu.SemaphoreType.DMA((Z - 1,)),
            pltpu.SemaphoreType.DMA((Z - 1,)),
            pltpu.SemaphoreType.DMA((Z - 1,)),
            pltpu.SemaphoreType.DMA((Z - 1,)),
        ],
        compiler_params=pltpu.CompilerParams(collective_id=0),
    )(Q, K, V)


# device time: 94878 ns/iter; 1.7387x vs baseline; 1.7387x over previous
import jax
import jax.numpy as jnp
from jax import lax
from jax.experimental import pallas as pl
from jax.experimental.pallas import tpu as pltpu

Z = 4


def kernel(Q, K, V):
    b, sq, nh, d = Q.shape
    scale = d ** -0.5

    qT = (jnp.transpose(Q, (0, 2, 1, 3)) * scale).astype(jnp.bfloat16)
    kT = jnp.transpose(K, (0, 2, 1, 3)).astype(jnp.bfloat16)
    vT = jnp.transpose(V, (0, 2, 1, 3)).astype(jnp.bfloat16)

    def body(q_ref, k_ref, v_ref, oT, kfull, vfull,
             ksend, krecv, vsend, vrecv):
        my_x = lax.axis_index("x")
        my_y = lax.axis_index("y")
        my_z = lax.axis_index("z")
        right = lax.rem(my_z + 1, Z)
        left = lax.rem(my_z + Z - 1, Z)

        barrier_sem = pltpu.get_barrier_semaphore()
        for nbr in (left, right):
            pl.semaphore_signal(
                barrier_sem, inc=1,
                device_id=(my_x, my_y, nbr),
                device_id_type=pl.DeviceIdType.MESH,
            )
        pl.semaphore_wait(barrier_sem, 2)

        kfull[my_z] = k_ref[...]
        vfull[my_z] = v_ref[...]

        for hop in range(Z - 1):
            slot = lax.rem(my_z - hop + Z, Z)
            k_rdma = pltpu.make_async_remote_copy(
                src_ref=kfull.at[slot],
                dst_ref=kfull.at[slot],
                send_sem=ksend.at[hop],
                recv_sem=krecv.at[hop],
                device_id=(my_x, my_y, right),
                device_id_type=pl.DeviceIdType.MESH,
            )
            v_rdma = pltpu.make_async_remote_copy(
                src_ref=vfull.at[slot],
                dst_ref=vfull.at[slot],
                send_sem=vsend.at[hop],
                recv_sem=vrecv.at[hop],
                device_id=(my_x, my_y, right),
                device_id_type=pl.DeviceIdType.MESH,
            )
            k_rdma.start()
            v_rdma.start()
            k_rdma.wait()
            v_rdma.wait()

        for bi in range(b):
            def head_body(hi, _):
                q = q_ref[bi, hi]
                k = kfull[:, bi, hi].reshape(Z * sq, d)
                v = vfull[:, bi, hi].reshape(Z * sq, d)
                s = lax.dot_general(
                    q, k, (((1,), (1,)), ((), ())),
                    preferred_element_type=jnp.float32,
                )
                m = jnp.max(s, axis=1, keepdims=True)
                p = jnp.exp(s - m)
                p = (p / jnp.sum(p, axis=1, keepdims=True)).astype(jnp.bfloat16)
                oT[bi, hi] = lax.dot_general(
                    p, v, (((1,), (0,)), ((), ())),
                    preferred_element_type=jnp.float32,
                )
                return 0

            lax.fori_loop(0, nh, head_body, 0)

    oT = pl.pallas_call(
        body,
        out_shape=jax.ShapeDtypeStruct((b, nh, sq, d), jnp.float32),
        in_specs=[pl.BlockSpec(memory_space=pltpu.VMEM)] * 3,
        out_specs=pl.BlockSpec(memory_space=pltpu.VMEM),
        scratch_shapes=[
            pltpu.VMEM((Z, b, nh, sq, d), jnp.bfloat16),
            pltpu.VMEM((Z, b, nh, sq, d), jnp.bfloat16),
            pltpu.SemaphoreType.DMA((Z - 1,)),
            pltpu.SemaphoreType.DMA((Z - 1,)),
            pltpu.SemaphoreType.DMA((Z - 1,)),
            pltpu.SemaphoreType.DMA((Z - 1,)),
        ],
        compiler_params=pltpu.CompilerParams(collective_id=0),
    )(qT, kT, vT)

    return jnp.transpose(oT, (0, 2, 1, 3))
